# Initial kernel scaffold; baseline (speedup 1.0000x reference)
import jax
import jax.numpy as jnp
from jax.experimental import pallas as pl


def kernel(x, edge_index, edge_attr, W1, b1, W2, b2, Ws, bs, Wn, bn):
    raise NotImplementedError("write your pallas kernel here")


# speedup vs baseline: 2.0967x; 2.0967x over previous
import functools

import jax
import jax.numpy as jnp
from jax import lax
from jax.experimental import pallas as pl
from jax.experimental.pallas import tpu as pltpu
from jax.experimental.pallas import tpu_sc as plsc

NC = 2
NS = 16
L = 16
NW = NC * NS

C = 128


def _edge_mlp_body(ea_ref, w1_ref, b1_ref, w2_ref, b2_ref, o_ref):
    h = jnp.dot(ea_ref[...], w1_ref[...], preferred_element_type=jnp.float32)
    h = jnp.maximum(h + b1_ref[...], 0.0)
    o_ref[...] = (
        jnp.dot(h, w2_ref[...], preferred_element_type=jnp.float32) + b2_ref[...]
    )


def _out_body(x_ref, p_ref, ws_ref, wn_ref, bs_ref, bn_ref, o_ref):
    agg = p_ref[0] + p_ref[1]
    acc = jnp.dot(x_ref[...], ws_ref[...], preferred_element_type=jnp.float32)
    acc += jnp.dot(agg, wn_ref[...], preferred_element_type=jnp.float32)
    o_ref[...] = jnp.maximum(acc + bs_ref[...] + bn_ref[...], 0.0)


def _make_sc_scatter(n_pad, d, e_pad):
    ew = e_pad // NW
    nchunk = ew // C
    rows_per_tile = n_pad // NS
    mesh = plsc.VectorSubcoreMesh(
        core_axis_name="c", subcore_axis_name="s", num_cores=NC, num_subcores=NS
    )

    @functools.partial(
        pl.kernel,
        out_type=jax.ShapeDtypeStruct((NC, n_pad, d), jnp.float32),
        mesh=mesh,
        scratch_types=[
            pltpu.VMEM((C,), jnp.int32),
            pltpu.VMEM((C,), jnp.int32),
            pltpu.VMEM((C, d), jnp.float32),
            pltpu.VMEM((C, d), jnp.float32),
            pltpu.VMEM_SHARED((n_pad, d), jnp.float32),
            pltpu.SemaphoreType.DMA,
        ],
    )
    def sc_fn(x_hbm, src_hbm, dst_hbm, w_hbm, part_hbm,
              src_v, dst_v, rows_v, w_v, agg_sh, sem):
        cid = lax.axis_index("c")
        sid = lax.axis_index("s")
        wid = sid * NC + cid
        gpd = d // L

        zero = jnp.zeros((L,), jnp.float32)

        def zrow(r, _):
            for g in range(gpd):
                rows_v[r, pl.ds(g * L, L)] = zero
            return 0

        lax.fori_loop(0, C, zrow, 0)
        r0 = sid * rows_per_tile
        off = 0
        while off < rows_per_tile:
            n = min(C, rows_per_tile - off)
            pltpu.sync_copy(rows_v.at[pl.ds(0, n)],
                            agg_sh.at[pl.ds(r0 + off, n)])
            off += n
        plsc.subcore_barrier()

        def chunk(j, _):
            base = wid * ew + j * C
            pltpu.sync_copy(src_hbm.at[pl.ds(base, C)], src_v)
            pltpu.sync_copy(dst_hbm.at[pl.ds(base, C)], dst_v)
            pltpu.async_copy(x_hbm.at[src_v], rows_v, sem).wait()
            pltpu.sync_copy(w_hbm.at[pl.ds(base, C)], w_v)

            def row(r, _):
                for g in range(gpd):
                    sl = pl.ds(g * L, L)
                    rows_v[r, sl] = rows_v[r, sl] * w_v[r, sl]
                return 0

            lax.fori_loop(0, C, row, 0)
            pltpu.sync_copy(rows_v, agg_sh.at[dst_v], add=True)
            return 0

        lax.fori_loop(0, nchunk, chunk, 0)
        plsc.subcore_barrier()
        pltpu.sync_copy(agg_sh.at[pl.ds(r0, rows_per_tile)],
                        part_hbm.at[cid, pl.ds(r0, rows_per_tile)])

    return sc_fn


def kernel(x, edge_index, edge_attr, W1, b1, W2, b2, Ws, bs, Wn, bn):
    n, d = x.shape
    e, ed = edge_attr.shape

    e_pad = ((e + NW * C - 1) // (NW * C)) * (NW * C)
    pad = e_pad - e
    n_pad = (n // (NS * 8) + 1) * (NS * 8)

    src = edge_index[0]
    dst = edge_index[1]
    if pad:
        src = jnp.concatenate([src, jnp.zeros((pad,), jnp.int32)])
        dst = jnp.concatenate([dst, jnp.full((pad,), n, jnp.int32)])
        ea = jnp.concatenate([edge_attr, jnp.zeros((pad, ed), edge_attr.dtype)])
    else:
        ea = edge_attr

    be = 2048
    grid_e = e_pad // be
    w = pl.pallas_call(
        _edge_mlp_body,
        grid=(grid_e,),
        in_specs=[
            pl.BlockSpec((be, ed), lambda i: (i, 0)),
            pl.BlockSpec((ed, d), lambda i: (0, 0)),
            pl.BlockSpec((1, d), lambda i: (0, 0)),
            pl.BlockSpec((d, d), lambda i: (0, 0)),
            pl.BlockSpec((1, d), lambda i: (0, 0)),
        ],
        out_specs=pl.BlockSpec((be, d), lambda i: (i, 0)),
        out_shape=jax.ShapeDtypeStruct((e_pad, d), jnp.float32),
    )(ea, W1, b1.reshape(1, d), W2, b2.reshape(1, d))

    parts = _make_sc_scatter(n_pad, d, e_pad)(x, src, dst, w)

    bn_rows = 1000
    grid_n = n // bn_rows
    out = pl.pallas_call(
        _out_body,
        grid=(grid_n,),
        in_specs=[
            pl.BlockSpec((bn_rows, d), lambda i: (i, 0)),
            pl.BlockSpec((NC, bn_rows, d), lambda i: (0, i, 0)),
            pl.BlockSpec((d, d), lambda i: (0, 0)),
            pl.BlockSpec((d, d), lambda i: (0, 0)),
            pl.BlockSpec((1, d), lambda i: (0, 0)),
            pl.BlockSpec((1, d), lambda i: (0, 0)),
        ],
        out_specs=pl.BlockSpec((bn_rows, d), lambda i: (i, 0)),
        out_shape=jax.ShapeDtypeStruct((n, d), jnp.float32),
    )(x, parts, Ws, Wn, bs.reshape(1, d), bn.reshape(1, d))
    return out


# speedup vs baseline: 2.7985x; 1.3347x over previous
import functools

import jax
import jax.numpy as jnp
from jax import lax
from jax.experimental import pallas as pl
from jax.experimental.pallas import tpu as pltpu
from jax.experimental.pallas import tpu_sc as plsc

NC = 2
NS = 16
L = 16
NW = NC * NS

C = 64


def _edge_mlp_body(ea_ref, w1_ref, b1_ref, w2_ref, b2_ref, o_ref):
    h = jnp.dot(ea_ref[...], w1_ref[...], preferred_element_type=jnp.float32)
    h = jnp.maximum(h + b1_ref[...], 0.0)
    o_ref[...] = (
        jnp.dot(h, w2_ref[...], preferred_element_type=jnp.float32) + b2_ref[...]
    )


def _out_body(x_ref, p_ref, ws_ref, wn_ref, bs_ref, bn_ref, o_ref):
    agg = p_ref[0] + p_ref[1]
    acc = jnp.dot(x_ref[...], ws_ref[...], preferred_element_type=jnp.float32)
    acc += jnp.dot(agg, wn_ref[...], preferred_element_type=jnp.float32)
    o_ref[...] = jnp.maximum(acc + bs_ref[...] + bn_ref[...], 0.0)


def _make_sc_scatter(n_pad, d, e_pad, trash_row):
    ew = e_pad // NW
    nchunk = ew // C
    half = nchunk // 2
    rows_per_tile = n_pad // NS
    gpd = d // L
    mesh = plsc.VectorSubcoreMesh(
        core_axis_name="c", subcore_axis_name="s", num_cores=NC, num_subcores=NS
    )

    @functools.partial(
        pl.kernel,
        out_type=jax.ShapeDtypeStruct((NC, n_pad, d), jnp.float32),
        mesh=mesh,
        scratch_types=[
            pltpu.VMEM((2, 2, C), jnp.int32),
            pltpu.VMEM((2, C, d), jnp.float32),
            pltpu.VMEM((2, C, d), jnp.float32),
            pltpu.VMEM((2, C, d), jnp.float32),
            pltpu.VMEM((2, C), jnp.int32),
            pltpu.VMEM_SHARED((n_pad, d), jnp.float32),
            pltpu.SemaphoreType.DMA,
            pltpu.SemaphoreType.DMA,
            pltpu.SemaphoreType.DMA,
            pltpu.SemaphoreType.DMA,
            pltpu.SemaphoreType.DMA,
            pltpu.SemaphoreType.DMA,
            pltpu.SemaphoreType.DMA,
            pltpu.SemaphoreType.DMA,
        ],
    )
    def sc_fn(x_hbm, idxg_hbm, w_hbm, part_hbm,
              idx_v, rows_v, w_v, msg_v, dstb_v, agg_sh,
              si0, si1, sg0, sg1, sw0, sw1, ss0, ss1):
        cid = lax.axis_index("c")
        sid = lax.axis_index("s")
        wid = sid * NC + cid
        g0 = wid * nchunk
        si = (si0, si1)
        sg = (sg0, sg1)
        sw = (sw0, sw1)
        ss = (ss0, ss1)

        zero = jnp.zeros((L,), jnp.float32)

        def zrow(r, _):
            for g in range(gpd):
                msg_v[0, r, pl.ds(g * L, L)] = zero
            return 0

        lax.fori_loop(0, C, zrow, 0)

        r0 = sid * rows_per_tile
        off = 0
        while off < rows_per_tile:
            nn = min(C, rows_per_tile - off)
            pltpu.sync_copy(msg_v.at[0].at[pl.ds(0, nn)],
                            agg_sh.at[pl.ds(r0 + off, nn)])
            off += nn
        plsc.subcore_barrier()

        def wait_idx(q):
            pltpu.make_async_copy(idxg_hbm.at[g0], idx_v.at[q], si[q]).wait()

        def wait_gw(p):
            pltpu.make_async_copy(x_hbm.at[idx_v.at[p, 0]], rows_v.at[p],
                                  sg[p]).wait()
            pltpu.make_async_copy(w_hbm.at[pl.ds(0, C)], w_v.at[p],
                                  sw[p]).wait()

        def drain_scatter(p):
            pltpu.make_async_copy(
                msg_v.at[p], agg_sh.at[dstb_v.at[p]], ss[p]).wait()

        def issue_gw(j, q):
            pltpu.async_copy(x_hbm.at[idx_v.at[q, 0]], rows_v.at[q], sg[q])
            pltpu.async_copy(w_hbm.at[pl.ds((g0 + j) * C, C)], w_v.at[q],
                             sw[q])

        def copy_dst(p):
            for g in range(C // L):
                sl = pl.ds(g * L, L)
                dstb_v[p, sl] = idx_v[p, 1, sl]

        def compute(p):
            def row(r2, _):
                for u in range(2):
                    r = 2 * r2 + u
                    for g in range(gpd):
                        sl = pl.ds(g * L, L)
                        msg_v[p, r, sl] = rows_v[p, r, sl] * w_v[p, r, sl]
                return 0

            lax.fori_loop(0, C // 2, row, 0)

        def chunk(j, p, q, drain, prefetch, sync_scatter):
            if prefetch:
                wait_idx(q)
                issue_gw(j + 1, q)
            wait_gw(p)
            if drain:
                drain_scatter(p)
            copy_dst(p)
            if prefetch:
                pltpu.async_copy(idxg_hbm.at[g0 + j + 2], idx_v.at[p], si[p])
            compute(p)
            if sync_scatter:
                pltpu.sync_copy(msg_v.at[p], agg_sh.at[dstb_v.at[p]],
                                add=True)
            else:
                pltpu.async_copy(msg_v.at[p], agg_sh.at[dstb_v.at[p]], ss[p],
                                 add=True)

        pltpu.sync_copy(idxg_hbm.at[g0], idx_v.at[0])
        pltpu.async_copy(idxg_hbm.at[g0 + 1], idx_v.at[1], si1)
        issue_gw(0, 0)

        chunk(0, 0, 1, drain=False, prefetch=True, sync_scatter=False)
        chunk(1, 1, 0, drain=False, prefetch=True, sync_scatter=False)

        def body(t, _):
            chunk(2 * t, 0, 1, drain=True, prefetch=True, sync_scatter=False)
            chunk(2 * t + 1, 1, 0, drain=True, prefetch=True,
                  sync_scatter=False)
            return 0

        lax.fori_loop(1, half - 1, body, 0)

        chunk(nchunk - 2, 0, 1, drain=True, prefetch=False, sync_scatter=True)
        pltpu.make_async_copy(idxg_hbm.at[g0], idx_v.at[1], si1).wait()
        issue_gw(nchunk - 1, 1)
        chunk(nchunk - 1, 1, 0, drain=True, prefetch=False, sync_scatter=True)

        plsc.subcore_barrier()
        pltpu.sync_copy(agg_sh.at[pl.ds(r0, rows_per_tile)],
                        part_hbm.at[cid, pl.ds(r0, rows_per_tile)])

    return sc_fn


def kernel(x, edge_index, edge_attr, W1, b1, W2, b2, Ws, bs, Wn, bn):
    n, d = x.shape
    e, ed = edge_attr.shape

    unit = 2 * NW * C
    e_pad = ((e + unit - 1) // unit) * unit
    pad = e_pad - e
    n_pad = (n // (NS * 8) + 1) * (NS * 8)

    src = edge_index[0]
    dst = edge_index[1]
    if pad:
        src = jnp.concatenate([src, jnp.zeros((pad,), jnp.int32)])
        dst = jnp.concatenate([dst, jnp.full((pad,), n, jnp.int32)])
        ea = jnp.concatenate([edge_attr, jnp.zeros((pad, ed), edge_attr.dtype)])
    else:
        ea = edge_attr
    idxg = jnp.stack([src.reshape(-1, C), dst.reshape(-1, C)], axis=1)

    be = 2048
    grid_e = e_pad // be
    w = pl.pallas_call(
        _edge_mlp_body,
        grid=(grid_e,),
        in_specs=[
            pl.BlockSpec((be, ed), lambda i: (i, 0)),
            pl.BlockSpec((ed, d), lambda i: (0, 0)),
            pl.BlockSpec((1, d), lambda i: (0, 0)),
            pl.BlockSpec((d, d), lambda i: (0, 0)),
            pl.BlockSpec((1, d), lambda i: (0, 0)),
        ],
        out_specs=pl.BlockSpec((be, d), lambda i: (i, 0)),
        out_shape=jax.ShapeDtypeStruct((e_pad, d), jnp.float32),
    )(ea, W1, b1.reshape(1, d), W2, b2.reshape(1, d))

    parts = _make_sc_scatter(n_pad, d, e_pad, n)(x, idxg, w)

    bn_rows = 1000
    grid_n = n // bn_rows
    out = pl.pallas_call(
        _out_body,
        grid=(grid_n,),
        in_specs=[
            pl.BlockSpec((bn_rows, d), lambda i: (i, 0)),
            pl.BlockSpec((NC, bn_rows, d), lambda i: (0, i, 0)),
            pl.BlockSpec((d, d), lambda i: (0, 0)),
            pl.BlockSpec((d, d), lambda i: (0, 0)),
            pl.BlockSpec((1, d), lambda i: (0, 0)),
            pl.BlockSpec((1, d), lambda i: (0, 0)),
        ],
        out_specs=pl.BlockSpec((bn_rows, d), lambda i: (i, 0)),
        out_shape=jax.ShapeDtypeStruct((n, d), jnp.float32),
    )(x, parts, Ws, Wn, bs.reshape(1, d), bn.reshape(1, d))
    return out


# speedup vs baseline: 3.4439x; 1.2306x over previous
import functools

import jax
import jax.numpy as jnp
from jax import lax
from jax.experimental import pallas as pl
from jax.experimental.pallas import tpu as pltpu
from jax.experimental.pallas import tpu_sc as plsc

NC = 2
NS = 16
L = 16
NW = NC * NS

C = 64


def _edge_mlp_body(ea_ref, w1_ref, b1_ref, w2_ref, b2_ref, o_ref):
    ea = ea_ref[...].astype(jnp.bfloat16)
    h = jnp.dot(ea, w1_ref[...], preferred_element_type=jnp.float32)
    h = jnp.maximum(h + b1_ref[...], 0.0).astype(jnp.bfloat16)
    o_ref[...] = (
        jnp.dot(h, w2_ref[...], preferred_element_type=jnp.float32) + b2_ref[...]
    )


def _out_body(x_ref, p_ref, ws_ref, wn_ref, bs_ref, bn_ref, o_ref):
    agg = p_ref[0] + p_ref[1]
    acc = jnp.dot(x_ref[...], ws_ref[...], preferred_element_type=jnp.float32)
    acc += jnp.dot(agg, wn_ref[...], preferred_element_type=jnp.float32)
    o_ref[...] = jnp.maximum(acc + bs_ref[...] + bn_ref[...], 0.0)


def _make_sc_scatter(n_pad, d, e_pad, trash_row):
    ew = e_pad // NW
    nchunk = ew // C
    half = nchunk // 2
    rows_per_tile = n_pad // NS
    gpd = d // L
    mesh = plsc.VectorSubcoreMesh(
        core_axis_name="c", subcore_axis_name="s", num_cores=NC, num_subcores=NS
    )

    @functools.partial(
        pl.kernel,
        out_type=jax.ShapeDtypeStruct((NC, n_pad, d), jnp.float32),
        mesh=mesh,
        scratch_types=[
            pltpu.VMEM((2, 2, C), jnp.int32),
            pltpu.VMEM((2, C, d), jnp.float32),
            pltpu.VMEM((2, C, d), jnp.float32),
            pltpu.VMEM((2, C, d), jnp.float32),
            pltpu.VMEM((2, C), jnp.int32),
            pltpu.VMEM_SHARED((n_pad, d), jnp.float32),
            pltpu.SemaphoreType.DMA,
            pltpu.SemaphoreType.DMA,
            pltpu.SemaphoreType.DMA,
            pltpu.SemaphoreType.DMA,
            pltpu.SemaphoreType.DMA,
            pltpu.SemaphoreType.DMA,
            pltpu.SemaphoreType.DMA,
            pltpu.SemaphoreType.DMA,
        ],
    )
    def sc_fn(x_hbm, idxg_hbm, w_hbm, part_hbm,
              idx_v, rows_v, w_v, msg_v, dstb_v, agg_sh,
              si0, si1, sg0, sg1, sw0, sw1, ss0, ss1):
        cid = lax.axis_index("c")
        sid = lax.axis_index("s")
        wid = sid * NC + cid
        g0 = wid * nchunk
        si = (si0, si1)
        sg = (sg0, sg1)
        sw = (sw0, sw1)
        ss = (ss0, ss1)

        zero = jnp.zeros((L,), jnp.float32)

        def zrow(r, _):
            for g in range(gpd):
                msg_v[0, r, pl.ds(g * L, L)] = zero
            return 0

        lax.fori_loop(0, C, zrow, 0)

        r0 = sid * rows_per_tile
        off = 0
        while off < rows_per_tile:
            nn = min(C, rows_per_tile - off)
            pltpu.sync_copy(msg_v.at[0].at[pl.ds(0, nn)],
                            agg_sh.at[pl.ds(r0 + off, nn)])
            off += nn
        plsc.subcore_barrier()

        def wait_idx(q):
            pltpu.make_async_copy(idxg_hbm.at[g0], idx_v.at[q], si[q]).wait()

        def wait_gw(p):
            pltpu.make_async_copy(x_hbm.at[idx_v.at[p, 0]], rows_v.at[p],
                                  sg[p]).wait()
            pltpu.make_async_copy(w_hbm.at[pl.ds(0, C)], w_v.at[p],
                                  sw[p]).wait()

        def drain_scatter(p):
            pltpu.make_async_copy(
                msg_v.at[p], agg_sh.at[dstb_v.at[p]], ss[p]).wait()

        def issue_gw(j, q):
            pltpu.async_copy(x_hbm.at[idx_v.at[q, 0]], rows_v.at[q], sg[q])
            pltpu.async_copy(w_hbm.at[pl.ds((g0 + j) * C, C)], w_v.at[q],
                             sw[q])

        def copy_dst(p):
            for g in range(C // L):
                sl = pl.ds(g * L, L)
                dstb_v[p, sl] = idx_v[p, 1, sl]

        def compute(p):
            def row(r2, _):
                for u in range(2):
                    r = 2 * r2 + u
                    for g in range(gpd):
                        sl = pl.ds(g * L, L)
                        msg_v[p, r, sl] = rows_v[p, r, sl] * w_v[p, r, sl]
                return 0

            lax.fori_loop(0, C // 2, row, 0)

        def chunk(j, p, q, drain, prefetch, sync_scatter):
            if prefetch:
                wait_idx(q)
                issue_gw(j + 1, q)
            wait_gw(p)
            if drain:
                drain_scatter(p)
            copy_dst(p)
            if prefetch:
                pltpu.async_copy(idxg_hbm.at[g0 + j + 2], idx_v.at[p], si[p])
            compute(p)
            if sync_scatter:
                pltpu.sync_copy(msg_v.at[p], agg_sh.at[dstb_v.at[p]],
                                add=True)
            else:
                pltpu.async_copy(msg_v.at[p], agg_sh.at[dstb_v.at[p]], ss[p],
                                 add=True)

        pltpu.sync_copy(idxg_hbm.at[g0], idx_v.at[0])
        pltpu.async_copy(idxg_hbm.at[g0 + 1], idx_v.at[1], si1)
        issue_gw(0, 0)

        chunk(0, 0, 1, drain=False, prefetch=True, sync_scatter=False)
        chunk(1, 1, 0, drain=False, prefetch=True, sync_scatter=False)

        def body(t, _):
            chunk(2 * t, 0, 1, drain=True, prefetch=True, sync_scatter=False)
            chunk(2 * t + 1, 1, 0, drain=True, prefetch=True,
                  sync_scatter=False)
            return 0

        lax.fori_loop(1, half - 1, body, 0)

        chunk(nchunk - 2, 0, 1, drain=True, prefetch=False, sync_scatter=True)
        pltpu.make_async_copy(idxg_hbm.at[g0], idx_v.at[1], si1).wait()
        issue_gw(nchunk - 1, 1)
        chunk(nchunk - 1, 1, 0, drain=True, prefetch=False, sync_scatter=True)

        plsc.subcore_barrier()
        pltpu.sync_copy(agg_sh.at[pl.ds(r0, rows_per_tile)],
                        part_hbm.at[cid, pl.ds(r0, rows_per_tile)])

    return sc_fn


def kernel(x, edge_index, edge_attr, W1, b1, W2, b2, Ws, bs, Wn, bn):
    n, d = x.shape
    e, ed = edge_attr.shape

    unit = 2 * NW * C
    e_pad = ((e + unit - 1) // unit) * unit
    pad = e_pad - e
    n_pad = (n // (NS * 8) + 1) * (NS * 8)

    src = edge_index[0]
    dst = edge_index[1]
    if pad:
        ar = jnp.arange(pad, dtype=jnp.int32)
        src = jnp.concatenate([src, ar % n])
        dst = jnp.concatenate([dst, n + ar % (n_pad - n)])
    idxg = jnp.stack([src.reshape(-1, C), dst.reshape(-1, C)], axis=1)

    be = 1280
    assert e % be == 0
    grid_e = e // be
    w = pl.pallas_call(
        _edge_mlp_body,
        grid=(grid_e,),
        in_specs=[
            pl.BlockSpec((be, ed), lambda i: (i, 0)),
            pl.BlockSpec((ed, d), lambda i: (0, 0)),
            pl.BlockSpec((1, d), lambda i: (0, 0)),
            pl.BlockSpec((d, d), lambda i: (0, 0)),
            pl.BlockSpec((1, d), lambda i: (0, 0)),
        ],
        out_specs=pl.BlockSpec((be, d), lambda i: (i, 0)),
        out_shape=jax.ShapeDtypeStruct((e_pad, d), jnp.float32),
    )(edge_attr, W1.astype(jnp.bfloat16), b1.reshape(1, d),
      W2.astype(jnp.bfloat16), b2.reshape(1, d))

    parts = _make_sc_scatter(n_pad, d, e_pad, n)(x, idxg, w)

    bn_rows = 1000
    grid_n = n // bn_rows
    out = pl.pallas_call(
        _out_body,
        grid=(grid_n,),
        in_specs=[
            pl.BlockSpec((bn_rows, d), lambda i: (i, 0)),
            pl.BlockSpec((NC, bn_rows, d), lambda i: (0, i, 0)),
            pl.BlockSpec((d, d), lambda i: (0, 0)),
            pl.BlockSpec((d, d), lambda i: (0, 0)),
            pl.BlockSpec((1, d), lambda i: (0, 0)),
            pl.BlockSpec((1, d), lambda i: (0, 0)),
        ],
        out_specs=pl.BlockSpec((bn_rows, d), lambda i: (i, 0)),
        out_shape=jax.ShapeDtypeStruct((n, d), jnp.float32),
    )(x, parts, Ws, Wn, bs.reshape(1, d), bn.reshape(1, d))
    return out
